# Initial kernel scaffold; baseline (speedup 1.0000x reference)
import jax
import jax.numpy as jnp
from jax.experimental import pallas as pl


def kernel(X, Y, Y_m, mask, W_e, b_e, elem_emb, ctx_W, ctx_b, L_W1, L_b1, L_W2, L_b2, L_W3, L_b3, L_n1g, L_n1b, L_Wi, L_bi, L_Wo, L_bo, L_n2g, L_n2b, fln_g, fln_b, Wd, bd, Y_t, R_idx):
    raise NotImplementedError("write your pallas kernel here")


# speedup vs baseline: 2.6309x; 2.6309x over previous
import jax
import jax.numpy as jnp
from jax import lax
from jax.experimental import pallas as pl
from jax.experimental.pallas import tpu as pltpu

B = 2
N = 1024
M = 300
K = 48
C = 25
H = 128
CS = 384
NE = 64
NL = 3

RK = 64
RC = 64
RE = 16
RL = 16
RF = 256

_F = jnp.float32
_I = jnp.int32


def _centers(rows):
    i = lax.broadcasted_iota(_I, (rows, 16), 1).astype(_F)
    return 2.0 + i * (20.0 / 15.0)


def _rbf(d, rows):
    z = (d - _centers(rows)) / 1.25
    return jnp.exp(-(z * z))


def _knn_body(ca_ref, caT_ref, idx_ref):
    nb = pl.program_id(1)
    ca = ca_ref[...]
    x = ca[:, 0:1]
    y = ca[:, 1:2]
    z = ca[:, 2:3]
    caT = caT_ref[...][0]
    dx = x - caT[0:1, :]
    dy = y - caT[1:2, :]
    dz = z - caT[2:3, :]
    d = jnp.sqrt(dx * dx + dy * dy + dz * dz + 1e-6)
    rows = nb * RK + lax.broadcasted_iota(_I, (RK, N), 0)
    cols = lax.broadcasted_iota(_I, (RK, N), 1)
    d = jnp.where(rows == cols, d + 1e6, d)
    colsK = lax.broadcasted_iota(_I, (RK, K), 1)
    idxm = jnp.zeros((RK, K), _I)
    for k in range(K):
        m = jnp.min(d, axis=1, keepdims=True)
        am = jnp.min(jnp.where(d == m, cols, N), axis=1, keepdims=True)
        idxm = jnp.where(colsK == k, am, idxm)
        d = jnp.where(cols == am, 3e38, d)
    idx_ref[...] = idxm


def _ctx_body(ca_ref, yT_ref, ym_ref, ytc_ref, emb_ref, cw_ref, cb_ref, out_ref):
    ca = ca_ref[...]
    x = ca[:, 0:1]
    y = ca[:, 1:2]
    z = ca[:, 2:3]
    yT = yT_ref[...][0]
    dx = x - yT[0:1, :]
    dy = y - yT[1:2, :]
    dz = z - yT[2:3, :]
    d = jnp.sqrt(dx * dx + dy * dy + dz * dz + 1e-6)
    d = jnp.where(ym_ref[...][0] > 0, d, 1e6)
    colsM = lax.broadcasted_iota(_I, (RC, M), 1)
    ohy = (ytc_ref[...][0] == lax.broadcasted_iota(_I, (M, NE), 1)).astype(_F)
    emby = jnp.dot(ohy, emb_ref[...], preferred_element_type=_F)
    cw = cw_ref[...]
    cb = cb_ref[...]
    acc = jnp.zeros((RC, H), _F)
    for c in range(C):
        m = jnp.min(d, axis=1, keepdims=True)
        am = jnp.min(jnp.where(d == m, colsM, M), axis=1, keepdims=True)
        d = jnp.where(colsM == am, 3e6, d)
        r = _rbf(m, RC)
        oh = (colsM == am).astype(_F)
        e_c = jnp.dot(oh, emby, preferred_element_type=_F)
        pre = (jnp.dot(r, cw[0:16, :], preferred_element_type=_F)
               + jnp.dot(e_c, cw[16:32, :], preferred_element_type=_F) + cb)
        acc = acc + jax.nn.gelu(pre)
    out_ref[...] = acc / C


def _edge_body(x12f_ref, x12b_ref, idxf_ref, we_ref, be_ref, out_ref, feats_ref):
    nb = pl.program_id(1)
    EK = RE * K
    colsN = lax.broadcasted_iota(_I, (EK, N), 1)
    idxf = idxf_ref[...]
    oh = (idxf == colsN).astype(_F)
    xn = jnp.dot(oh, x12f_ref[...], preferred_element_type=_F)
    rep = ((lax.broadcasted_iota(_I, (EK, RE), 0) // K)
           == lax.broadcasted_iota(_I, (EK, RE), 1)).astype(_F)
    xs = jnp.dot(rep, x12b_ref[...], preferred_element_type=_F)
    for a in range(4):
        xa = xs[:, 3 * a:3 * a + 3]
        for b in range(4):
            xb = xn[:, 3 * b:3 * b + 3]
            df = xa - xb
            d2 = jnp.sum(df * df, axis=1, keepdims=True)
            d = jnp.sqrt(d2 + 1e-6)
            p = a * 4 + b
            feats_ref[:, 16 * p:16 * (p + 1)] = _rbf(d, EK)
    rowid = nb * RE + lax.broadcasted_iota(_I, (EK, 1), 0) // K
    off = jnp.clip(idxf - rowid, -32, 32) + 32
    ohoff = (off == lax.broadcasted_iota(_I, (EK, 65), 1)).astype(_F)
    we = we_ref[...]
    e = (jnp.dot(feats_ref[...], we[0:256, :], preferred_element_type=_F)
         + jnp.dot(ohoff, we[256:321, :], preferred_element_type=_F)
         + be_ref[...])
    out_ref[...] = e


def _ln(h, g, b):
    mu = jnp.mean(h, axis=1, keepdims=True)
    v = jnp.mean((h - mu) ** 2, axis=1, keepdims=True)
    return (h - mu) / jnp.sqrt(v + 1e-5) * g + b


def _layer_body(hvf_ref, hvb_ref, eb_ref, idxf_ref, w1_ref, b1_ref, w2_ref,
                b2_ref, w3_ref, b3_ref, g1_ref, bb1_ref, wi_ref, bi_ref,
                wo_ref, bo_ref, g2_ref, bb2_ref, mk_ref, out_ref):
    EK = RL * K
    colsN = lax.broadcasted_iota(_I, (EK, N), 1)
    oh = (idxf_ref[...] == colsN).astype(_F)
    hnb = jnp.dot(oh, hvf_ref[...], preferred_element_type=_F)
    rep = ((lax.broadcasted_iota(_I, (EK, RL), 0) // K)
           == lax.broadcasted_iota(_I, (EK, RL), 1)).astype(_F)
    hvb = hvb_ref[...]
    hvrep = jnp.dot(rep, hvb, preferred_element_type=_F)
    w1 = w1_ref[...]
    pre = (jnp.dot(hvrep, w1[0:H, :], preferred_element_type=_F)
           + jnp.dot(eb_ref[...], w1[H:2 * H, :], preferred_element_type=_F)
           + jnp.dot(hnb, w1[2 * H:3 * H, :], preferred_element_type=_F)
           + b1_ref[...])
    m = jax.nn.gelu(pre)
    m = jax.nn.gelu(jnp.dot(m, w2_ref[...], preferred_element_type=_F) + b2_ref[...])
    m = jnp.dot(m, w3_ref[...], preferred_element_type=_F) + b3_ref[...]
    summat = (lax.broadcasted_iota(_I, (RL, EK), 0)
              == lax.broadcasted_iota(_I, (RL, EK), 1) // K).astype(_F)
    msum = jnp.dot(summat, m, preferred_element_type=_F)
    h = hvb + msum / K
    hn = _ln(h, g1_ref[...], bb1_ref[...])
    ff = (jnp.dot(jax.nn.gelu(jnp.dot(hn, wi_ref[...], preferred_element_type=_F)
                              + bi_ref[...]),
                  wo_ref[...], preferred_element_type=_F) + bo_ref[...])
    h2 = _ln(hn + ff, g2_ref[...], bb2_ref[...])
    out_ref[...] = h2 * mk_ref[...]


def _final_body(hv_ref, g_ref, b_ref, wd_ref, bd_ref, out_ref):
    hn = _ln(hv_ref[...], g_ref[...], b_ref[...])
    out_ref[...] = jnp.dot(hn, wd_ref[...], preferred_element_type=_F) + bd_ref[...]


def _full(shape):
    return pl.BlockSpec(shape, lambda b, nb: (0, 0))


def _perb(shape):
    return pl.BlockSpec(shape, lambda b, nb: (b, 0))


def _perb3(d1, d2):
    return pl.BlockSpec((1, d1, d2), lambda b, nb: (b, 0, 0))


def _blk(shape, nblk):
    return pl.BlockSpec(shape, lambda b, nb: (b * nblk + nb, 0))


def kernel(X, Y, Y_m, mask, W_e, b_e, elem_emb, ctx_W, ctx_b, L_W1, L_b1,
           L_W2, L_b2, L_W3, L_b3, L_n1g, L_n1b, L_Wi, L_bi, L_Wo, L_bo,
           L_n2g, L_n2b, fln_g, fln_b, Wd, bd, Y_t, R_idx):
    Ca = X[:, :, 1, :]
    ca2 = Ca.reshape(B * N, 3)
    caT = jnp.swapaxes(Ca, 1, 2)
    x12 = X.reshape(B * N, 12)
    yT = jnp.swapaxes(Y, 1, 2)
    ym3 = Y_m.reshape(B, 1, M)
    ytc = Y_t.reshape(B, M, 1).astype(_I)
    maskc = mask.reshape(B * N, 1)

    idx = pl.pallas_call(
        _knn_body,
        grid=(B, N // RK),
        in_specs=[_blk((RK, 3), N // RK), _perb3(3, N)],
        out_specs=_blk((RK, K), N // RK),
        out_shape=jax.ShapeDtypeStruct((B * N, K), _I),
    )(ca2, caT)
    idxf = idx.reshape(B * N * K, 1)

    hv = pl.pallas_call(
        _ctx_body,
        grid=(B, N // RC),
        in_specs=[_blk((RC, 3), N // RC), _perb3(3, M), _perb3(1, M),
                  _perb3(M, 1), _full((NE, 16)), _full((32, H)),
                  _full((1, H))],
        out_specs=_blk((RC, H), N // RC),
        out_shape=jax.ShapeDtypeStruct((B * N, H), _F),
    )(ca2, yT, ym3, ytc, elem_emb, ctx_W, ctx_b.reshape(1, H))

    E = pl.pallas_call(
        _edge_body,
        grid=(B, N // RE),
        in_specs=[_perb((N, 12)), _blk((RE, 12), N // RE),
                  _blk((RE * K, 1), N // RE), _full((321, H)), _full((1, H))],
        out_specs=_blk((RE * K, H), N // RE),
        out_shape=jax.ShapeDtypeStruct((B * N * K, H), _F),
        scratch_shapes=[pltpu.VMEM((RE * K, 256), _F)],
    )(x12, x12, idxf, W_e, b_e.reshape(1, H))

    for l in range(NL):
        hv = pl.pallas_call(
            _layer_body,
            grid=(B, N // RL),
            in_specs=[_perb((N, H)), _blk((RL, H), N // RL),
                      _blk((RL * K, H), N // RL), _blk((RL * K, 1), N // RL),
                      _full((3 * H, H)), _full((1, H)), _full((H, H)),
                      _full((1, H)), _full((H, H)), _full((1, H)),
                      _full((1, H)), _full((1, H)), _full((H, 4 * H)),
                      _full((1, 4 * H)), _full((4 * H, H)), _full((1, H)),
                      _full((1, H)), _full((1, H)), _blk((RL, 1), N // RL)],
            out_specs=_blk((RL, H), N // RL),
            out_shape=jax.ShapeDtypeStruct((B * N, H), _F),
        )(hv, hv, E, idxf, L_W1[l], L_b1[l].reshape(1, H), L_W2[l],
          L_b2[l].reshape(1, H), L_W3[l], L_b3[l].reshape(1, H),
          L_n1g[l].reshape(1, H), L_n1b[l].reshape(1, H), L_Wi[l],
          L_bi[l].reshape(1, 4 * H), L_Wo[l], L_bo[l].reshape(1, H),
          L_n2g[l].reshape(1, H), L_n2b[l].reshape(1, H), maskc)

    out = pl.pallas_call(
        _final_body,
        grid=(B * N // RF, 1),
        in_specs=[_blk((RF, H), 1), _full((1, H)), _full((1, H)),
                  _full((H, CS)), _full((1, CS))],
        out_specs=_blk((RF, CS), 1),
        out_shape=jax.ShapeDtypeStruct((B * N, CS), _F),
    )(hv, fln_g.reshape(1, H), fln_b.reshape(1, H), Wd, bd.reshape(1, CS))
    return out.reshape(B, N, CS)


# speedup vs baseline: 4.0218x; 1.5286x over previous
import jax
import jax.numpy as jnp
from jax import lax
from jax.experimental import pallas as pl
from jax.experimental.pallas import tpu as pltpu

B = 2
N = 1024
M = 300
K = 48
C = 25
H = 128
CS = 384
NE = 64
NL = 3

RK = 64
RC = 64
RE = 16
RL = 16
RF = 256

_F = jnp.float32
_I = jnp.int32


def _centers(rows):
    i = lax.broadcasted_iota(_I, (rows, 16), 1).astype(_F)
    return 2.0 + i * (20.0 / 15.0)


def _rbf(d, rows):
    z = (d - _centers(rows)) / 1.25
    return jnp.exp(-(z * z))


def _knn_body(ca_ref, caT_ref, idx_ref):
    nb = pl.program_id(1)
    ca = ca_ref[...]
    x = ca[:, 0:1]
    y = ca[:, 1:2]
    z = ca[:, 2:3]
    caT = caT_ref[...][0]
    dx = x - caT[0:1, :]
    dy = y - caT[1:2, :]
    dz = z - caT[2:3, :]
    d = jnp.sqrt(dx * dx + dy * dy + dz * dz + 1e-6)
    rows = nb * RK + lax.broadcasted_iota(_I, (RK, N), 0)
    cols = lax.broadcasted_iota(_I, (RK, N), 1)
    d = jnp.where(rows == cols, d + 1e6, d)
    colsK = lax.broadcasted_iota(_I, (RK, K), 1)
    idxm = jnp.zeros((RK, K), _I)
    for k in range(K):
        m = jnp.min(d, axis=1, keepdims=True)
        am = jnp.min(jnp.where(d == m, cols, N), axis=1, keepdims=True)
        idxm = jnp.where(colsK == k, am, idxm)
        d = jnp.where(cols == am, 3e38, d)
    idx_ref[...] = idxm


def _ctx_body(ca_ref, yT_ref, ym_ref, ytc_ref, emb_ref, cw_ref, cb_ref, out_ref):
    ca = ca_ref[...]
    x = ca[:, 0:1]
    y = ca[:, 1:2]
    z = ca[:, 2:3]
    yT = yT_ref[...][0]
    dx = x - yT[0:1, :]
    dy = y - yT[1:2, :]
    dz = z - yT[2:3, :]
    d = jnp.sqrt(dx * dx + dy * dy + dz * dz + 1e-6)
    d = jnp.where(ym_ref[...][0] > 0, d, 1e6)
    colsM = lax.broadcasted_iota(_I, (RC, M), 1)
    ohy = (ytc_ref[...][0] == lax.broadcasted_iota(_I, (M, NE), 1)).astype(_F)
    emby = jnp.dot(ohy, emb_ref[...], preferred_element_type=_F)
    cw = cw_ref[...]
    cb = cb_ref[...]
    acc = jnp.zeros((RC, H), _F)
    for c in range(C):
        m = jnp.min(d, axis=1, keepdims=True)
        am = jnp.min(jnp.where(d == m, colsM, M), axis=1, keepdims=True)
        d = jnp.where(colsM == am, 3e6, d)
        r = _rbf(m, RC)
        oh = (colsM == am).astype(_F)
        e_c = jnp.dot(oh, emby, preferred_element_type=_F)
        pre = (jnp.dot(r, cw[0:16, :], preferred_element_type=_F)
               + jnp.dot(e_c, cw[16:32, :], preferred_element_type=_F) + cb)
        acc = acc + jax.nn.gelu(pre)
    out_ref[...] = acc / C


def _edge_body(x12f_ref, x12b_ref, idxf_ref, we_ref, be_ref, out_ref):
    nb = pl.program_id(1)
    EK = RE * K
    colsN = lax.broadcasted_iota(_I, (EK, N), 1)
    idxf = idxf_ref[...]
    oh = (idxf == colsN).astype(_F)
    xn = jnp.dot(oh, x12f_ref[...], preferred_element_type=_F)
    rep = ((lax.broadcasted_iota(_I, (EK, RE), 0) // K)
           == lax.broadcasted_iota(_I, (EK, RE), 1)).astype(_F)
    xs = jnp.dot(rep, x12b_ref[...], preferred_element_type=_F)
    d2 = None
    for c in range(3):
        r12 = lax.broadcasted_iota(_I, (12, 16), 0)
        p16 = lax.broadcasted_iota(_I, (12, 16), 1)
        sa = (r12 == 3 * (p16 // 4) + c).astype(_F)
        sb = (r12 == 3 * (p16 % 4) + c).astype(_F)
        t = (jnp.dot(xs, sa, preferred_element_type=_F)
             - jnp.dot(xn, sb, preferred_element_type=_F))
        t = t * t
        d2 = t if d2 is None else d2 + t
    d = jnp.sqrt(d2 + 1e-6)
    expand = (lax.broadcasted_iota(_I, (16, 256), 1) // 16
              == lax.broadcasted_iota(_I, (16, 256), 0)).astype(_F)
    d256 = jnp.dot(d, expand, preferred_element_type=_F)
    j256 = lax.broadcasted_iota(_I, (EK, 256), 1) % 16
    c256 = 2.0 + j256.astype(_F) * (20.0 / 15.0)
    z = (d256 - c256) * 0.8
    feats = jnp.exp(-(z * z))
    rowid = nb * RE + lax.broadcasted_iota(_I, (EK, 1), 0) // K
    off = jnp.clip(idxf - rowid, -32, 32) + 32
    ohoff = (off == lax.broadcasted_iota(_I, (EK, 65), 1)).astype(_F)
    we = we_ref[...]
    e = (jnp.dot(feats, we[0:256, :], preferred_element_type=_F)
         + jnp.dot(ohoff, we[256:321, :], preferred_element_type=_F)
         + be_ref[...])
    out_ref[...] = e


def _ln(h, g, b):
    mu = jnp.mean(h, axis=1, keepdims=True)
    v = jnp.mean((h - mu) ** 2, axis=1, keepdims=True)
    return (h - mu) / jnp.sqrt(v + 1e-5) * g + b


def _layer_body(hvf_ref, hvb_ref, eb_ref, idxf_ref, w1_ref, b1_ref, w2_ref,
                b2_ref, w3_ref, b3_ref, g1_ref, bb1_ref, wi_ref, bi_ref,
                wo_ref, bo_ref, g2_ref, bb2_ref, mk_ref, out_ref):
    EK = RL * K
    colsN = lax.broadcasted_iota(_I, (EK, N), 1)
    oh = (idxf_ref[...] == colsN).astype(jnp.bfloat16)
    hnb = jnp.dot(oh, hvf_ref[...].astype(jnp.bfloat16),
                  preferred_element_type=_F)
    rep = ((lax.broadcasted_iota(_I, (EK, RL), 0) // K)
           == lax.broadcasted_iota(_I, (EK, RL), 1)).astype(_F)
    hvb = hvb_ref[...]
    hvrep = jnp.dot(rep, hvb, preferred_element_type=_F)
    w1 = w1_ref[...]
    pre = (jnp.dot(hvrep, w1[0:H, :], preferred_element_type=_F)
           + jnp.dot(eb_ref[...], w1[H:2 * H, :], preferred_element_type=_F)
           + jnp.dot(hnb, w1[2 * H:3 * H, :], preferred_element_type=_F)
           + b1_ref[...])
    m = jax.nn.gelu(pre)
    m = jax.nn.gelu(jnp.dot(m, w2_ref[...], preferred_element_type=_F) + b2_ref[...])
    m = jnp.dot(m, w3_ref[...], preferred_element_type=_F) + b3_ref[...]
    summat = (lax.broadcasted_iota(_I, (RL, EK), 0)
              == lax.broadcasted_iota(_I, (RL, EK), 1) // K).astype(_F)
    msum = jnp.dot(summat, m, preferred_element_type=_F)
    h = hvb + msum / K
    hn = _ln(h, g1_ref[...], bb1_ref[...])
    ff = (jnp.dot(jax.nn.gelu(jnp.dot(hn, wi_ref[...], preferred_element_type=_F)
                              + bi_ref[...]),
                  wo_ref[...], preferred_element_type=_F) + bo_ref[...])
    h2 = _ln(hn + ff, g2_ref[...], bb2_ref[...])
    out_ref[...] = h2 * mk_ref[...]


def _final_body(hv_ref, g_ref, b_ref, wd_ref, bd_ref, out_ref):
    hn = _ln(hv_ref[...], g_ref[...], b_ref[...])
    out_ref[...] = jnp.dot(hn, wd_ref[...], preferred_element_type=_F) + bd_ref[...]


def _full(shape):
    return pl.BlockSpec(shape, lambda b, nb: (0, 0))


def _perb(shape):
    return pl.BlockSpec(shape, lambda b, nb: (b, 0))


def _perb3(d1, d2):
    return pl.BlockSpec((1, d1, d2), lambda b, nb: (b, 0, 0))


def _blk(shape, nblk):
    return pl.BlockSpec(shape, lambda b, nb: (b * nblk + nb, 0))


def kernel(X, Y, Y_m, mask, W_e, b_e, elem_emb, ctx_W, ctx_b, L_W1, L_b1,
           L_W2, L_b2, L_W3, L_b3, L_n1g, L_n1b, L_Wi, L_bi, L_Wo, L_bo,
           L_n2g, L_n2b, fln_g, fln_b, Wd, bd, Y_t, R_idx):
    Ca = X[:, :, 1, :]
    ca2 = Ca.reshape(B * N, 3)
    caT = jnp.swapaxes(Ca, 1, 2)
    x12 = X.reshape(B * N, 12)
    yT = jnp.swapaxes(Y, 1, 2)
    ym3 = Y_m.reshape(B, 1, M)
    ytc = Y_t.reshape(B, M, 1).astype(_I)
    maskc = mask.reshape(B * N, 1)

    idx = pl.pallas_call(
        _knn_body,
        grid=(B, N // RK),
        in_specs=[_blk((RK, 3), N // RK), _perb3(3, N)],
        out_specs=_blk((RK, K), N // RK),
        out_shape=jax.ShapeDtypeStruct((B * N, K), _I),
    )(ca2, caT)
    idxf = idx.reshape(B * N * K, 1)

    hv = pl.pallas_call(
        _ctx_body,
        grid=(B, N // RC),
        in_specs=[_blk((RC, 3), N // RC), _perb3(3, M), _perb3(1, M),
                  _perb3(M, 1), _full((NE, 16)), _full((32, H)),
                  _full((1, H))],
        out_specs=_blk((RC, H), N // RC),
        out_shape=jax.ShapeDtypeStruct((B * N, H), _F),
    )(ca2, yT, ym3, ytc, elem_emb, ctx_W, ctx_b.reshape(1, H))

    E = pl.pallas_call(
        _edge_body,
        grid=(B, N // RE),
        in_specs=[_perb((N, 12)), _blk((RE, 12), N // RE),
                  _blk((RE * K, 1), N // RE), _full((321, H)), _full((1, H))],
        out_specs=_blk((RE * K, H), N // RE),
        out_shape=jax.ShapeDtypeStruct((B * N * K, H), _F),
    )(x12, x12, idxf, W_e, b_e.reshape(1, H))

    for l in range(NL):
        hv = pl.pallas_call(
            _layer_body,
            grid=(B, N // RL),
            in_specs=[_perb((N, H)), _blk((RL, H), N // RL),
                      _blk((RL * K, H), N // RL), _blk((RL * K, 1), N // RL),
                      _full((3 * H, H)), _full((1, H)), _full((H, H)),
                      _full((1, H)), _full((H, H)), _full((1, H)),
                      _full((1, H)), _full((1, H)), _full((H, 4 * H)),
                      _full((1, 4 * H)), _full((4 * H, H)), _full((1, H)),
                      _full((1, H)), _full((1, H)), _blk((RL, 1), N // RL)],
            out_specs=_blk((RL, H), N // RL),
            out_shape=jax.ShapeDtypeStruct((B * N, H), _F),
        )(hv, hv, E, idxf, L_W1[l], L_b1[l].reshape(1, H), L_W2[l],
          L_b2[l].reshape(1, H), L_W3[l], L_b3[l].reshape(1, H),
          L_n1g[l].reshape(1, H), L_n1b[l].reshape(1, H), L_Wi[l],
          L_bi[l].reshape(1, 4 * H), L_Wo[l], L_bo[l].reshape(1, H),
          L_n2g[l].reshape(1, H), L_n2b[l].reshape(1, H), maskc)

    out = pl.pallas_call(
        _final_body,
        grid=(B * N // RF, 1),
        in_specs=[_blk((RF, H), 1), _full((1, H)), _full((1, H)),
                  _full((H, CS)), _full((1, CS))],
        out_specs=_blk((RF, CS), 1),
        out_shape=jax.ShapeDtypeStruct((B * N, CS), _F),
    )(hv, fln_g.reshape(1, H), fln_b.reshape(1, H), Wd, bd.reshape(1, CS))
    return out.reshape(B, N, CS)


# speedup vs baseline: 5.8132x; 1.4454x over previous
import jax
import jax.numpy as jnp
from jax import lax
from jax.experimental import pallas as pl
from jax.experimental.pallas import tpu as pltpu

B = 2
N = 1024
M = 300
K = 48
C = 25
H = 128
CS = 384
NE = 64
NL = 3

RK = 128
RC = 128
RE = 32
RL = 32
RF = 256

_F = jnp.float32
_I = jnp.int32


def _centers(rows):
    i = lax.broadcasted_iota(_I, (rows, 16), 1).astype(_F)
    return 2.0 + i * (20.0 / 15.0)


def _rbf(d, rows):
    z = (d - _centers(rows)) / 1.25
    return jnp.exp(-(z * z))


def _knn_body(ca_ref, caT_ref, idx_ref):
    nb = pl.program_id(1)
    ca = ca_ref[...]
    x = ca[:, 0:1]
    y = ca[:, 1:2]
    z = ca[:, 2:3]
    caT = caT_ref[...][0]
    dx = x - caT[0:1, :]
    dy = y - caT[1:2, :]
    dz = z - caT[2:3, :]
    d = jnp.sqrt(dx * dx + dy * dy + dz * dz + 1e-6)
    rows = nb * RK + lax.broadcasted_iota(_I, (RK, N), 0)
    cols = lax.broadcasted_iota(_I, (RK, N), 1)
    d = jnp.where(rows == cols, d + 1e6, d)
    colsK = lax.broadcasted_iota(_I, (RK, K), 1)
    idxm = jnp.zeros((RK, K), _I)
    for k in range(K):
        m = jnp.min(d, axis=1, keepdims=True)
        am = jnp.min(jnp.where(d == m, cols, N), axis=1, keepdims=True)
        idxm = jnp.where(colsK == k, am, idxm)
        d = jnp.where(cols == am, 3e38, d)
    idx_ref[...] = idxm


def _ctx_body(ca_ref, yT_ref, ym_ref, ytc_ref, emb_ref, cw_ref, cb_ref, out_ref):
    ca = ca_ref[...]
    x = ca[:, 0:1]
    y = ca[:, 1:2]
    z = ca[:, 2:3]
    yT = yT_ref[...][0]
    dx = x - yT[0:1, :]
    dy = y - yT[1:2, :]
    dz = z - yT[2:3, :]
    d = jnp.sqrt(dx * dx + dy * dy + dz * dz + 1e-6)
    d = jnp.where(ym_ref[...][0] > 0, d, 1e6)
    colsM = lax.broadcasted_iota(_I, (RC, M), 1)
    ohy = (ytc_ref[...][0] == lax.broadcasted_iota(_I, (M, NE), 1)).astype(_F)
    emby = jnp.dot(ohy, emb_ref[...], preferred_element_type=_F)
    cw = cw_ref[...]
    cb = cb_ref[...]
    acc = jnp.zeros((RC, H), _F)
    for c in range(C):
        m = jnp.min(d, axis=1, keepdims=True)
        am = jnp.min(jnp.where(d == m, colsM, M), axis=1, keepdims=True)
        d = jnp.where(colsM == am, 3e6, d)
        r = _rbf(m, RC)
        oh = (colsM == am).astype(_F)
        e_c = jnp.dot(oh, emby, preferred_element_type=_F)
        pre = (jnp.dot(r, cw[0:16, :], preferred_element_type=_F)
               + jnp.dot(e_c, cw[16:32, :], preferred_element_type=_F) + cb)
        acc = acc + jax.nn.gelu(pre)
    out_ref[...] = acc / C


def _edge_body(x12f_ref, x12b_ref, idxf_ref, we_ref, be_ref, out_ref):
    nb = pl.program_id(1)
    EK = RE * K
    colsN = lax.broadcasted_iota(_I, (EK, N), 1)
    idxf = idxf_ref[...]
    oh = (idxf == colsN).astype(_F)
    xn = jnp.dot(oh, x12f_ref[...], preferred_element_type=_F)
    rep = ((lax.broadcasted_iota(_I, (EK, RE), 0) // K)
           == lax.broadcasted_iota(_I, (EK, RE), 1)).astype(_F)
    xs = jnp.dot(rep, x12b_ref[...], preferred_element_type=_F)
    d2 = None
    for c in range(3):
        r12 = lax.broadcasted_iota(_I, (12, 16), 0)
        p16 = lax.broadcasted_iota(_I, (12, 16), 1)
        sa = (r12 == 3 * (p16 // 4) + c).astype(_F)
        sb = (r12 == 3 * (p16 % 4) + c).astype(_F)
        t = (jnp.dot(xs, sa, preferred_element_type=_F)
             - jnp.dot(xn, sb, preferred_element_type=_F))
        t = t * t
        d2 = t if d2 is None else d2 + t
    d = jnp.sqrt(d2 + 1e-6)
    expand = (lax.broadcasted_iota(_I, (16, 256), 1) // 16
              == lax.broadcasted_iota(_I, (16, 256), 0)).astype(_F)
    d256 = jnp.dot(d, expand, preferred_element_type=_F)
    j256 = lax.broadcasted_iota(_I, (EK, 256), 1) % 16
    c256 = 2.0 + j256.astype(_F) * (20.0 / 15.0)
    z = (d256 - c256) * 0.8
    feats = jnp.exp(-(z * z))
    rowid = nb * RE + lax.broadcasted_iota(_I, (EK, 1), 0) // K
    off = jnp.clip(idxf - rowid, -32, 32) + 32
    ohoff = (off == lax.broadcasted_iota(_I, (EK, 65), 1)).astype(_F)
    we = we_ref[...]
    e = (jnp.dot(feats, we[0:256, :], preferred_element_type=_F)
         + jnp.dot(ohoff, we[256:321, :], preferred_element_type=_F)
         + be_ref[...])
    out_ref[...] = e


def _ln(h, g, b):
    mu = jnp.mean(h, axis=1, keepdims=True)
    v = jnp.mean((h - mu) ** 2, axis=1, keepdims=True)
    return (h - mu) / jnp.sqrt(v + 1e-5) * g + b


def _layer_body(hvf_ref, hvb_ref, eb_ref, idxf_ref, w1_ref, b1_ref, w2_ref,
                b2_ref, w3_ref, b3_ref, g1_ref, bb1_ref, wi_ref, bi_ref,
                wo_ref, bo_ref, g2_ref, bb2_ref, mk_ref, out_ref):
    EK = RL * K
    colsN = lax.broadcasted_iota(_I, (EK, N), 1)
    oh = (idxf_ref[...] == colsN).astype(jnp.bfloat16)
    hnb = jnp.dot(oh, hvf_ref[...].astype(jnp.bfloat16),
                  preferred_element_type=_F)
    rep = ((lax.broadcasted_iota(_I, (EK, RL), 0) // K)
           == lax.broadcasted_iota(_I, (EK, RL), 1)).astype(_F)
    hvb = hvb_ref[...]
    hvrep = jnp.dot(rep, hvb, preferred_element_type=_F)
    w1 = w1_ref[...]
    pre = (jnp.dot(hvrep, w1[0:H, :], preferred_element_type=_F)
           + jnp.dot(eb_ref[...], w1[H:2 * H, :], preferred_element_type=_F)
           + jnp.dot(hnb, w1[2 * H:3 * H, :], preferred_element_type=_F)
           + b1_ref[...])
    m = jax.nn.gelu(pre)
    m = jax.nn.gelu(jnp.dot(m, w2_ref[...], preferred_element_type=_F) + b2_ref[...])
    m = jnp.dot(m, w3_ref[...], preferred_element_type=_F) + b3_ref[...]
    summat = (lax.broadcasted_iota(_I, (RL, EK), 0)
              == lax.broadcasted_iota(_I, (RL, EK), 1) // K).astype(_F)
    msum = jnp.dot(summat, m, preferred_element_type=_F)
    h = hvb + msum / K
    hn = _ln(h, g1_ref[...], bb1_ref[...])
    ff = (jnp.dot(jax.nn.gelu(jnp.dot(hn, wi_ref[...], preferred_element_type=_F)
                              + bi_ref[...]),
                  wo_ref[...], preferred_element_type=_F) + bo_ref[...])
    h2 = _ln(hn + ff, g2_ref[...], bb2_ref[...])
    out_ref[...] = h2 * mk_ref[...]


def _final_body(hv_ref, g_ref, b_ref, wd_ref, bd_ref, out_ref):
    hn = _ln(hv_ref[...], g_ref[...], b_ref[...])
    out_ref[...] = jnp.dot(hn, wd_ref[...], preferred_element_type=_F) + bd_ref[...]


def _full(shape):
    return pl.BlockSpec(shape, lambda b, nb: (0, 0))


def _perb(shape):
    return pl.BlockSpec(shape, lambda b, nb: (b, 0))


def _perb3(d1, d2):
    return pl.BlockSpec((1, d1, d2), lambda b, nb: (b, 0, 0))


def _blk(shape, nblk):
    return pl.BlockSpec(shape, lambda b, nb: (b * nblk + nb, 0))


def kernel(X, Y, Y_m, mask, W_e, b_e, elem_emb, ctx_W, ctx_b, L_W1, L_b1,
           L_W2, L_b2, L_W3, L_b3, L_n1g, L_n1b, L_Wi, L_bi, L_Wo, L_bo,
           L_n2g, L_n2b, fln_g, fln_b, Wd, bd, Y_t, R_idx):
    Ca = X[:, :, 1, :]
    ca2 = Ca.reshape(B * N, 3)
    caT = jnp.swapaxes(Ca, 1, 2)
    x12 = X.reshape(B * N, 12)
    yT = jnp.swapaxes(Y, 1, 2)
    ym3 = Y_m.reshape(B, 1, M)
    ytc = Y_t.reshape(B, M, 1).astype(_I)
    maskc = mask.reshape(B * N, 1)

    idx = pl.pallas_call(
        _knn_body,
        grid=(B, N // RK),
        in_specs=[_blk((RK, 3), N // RK), _perb3(3, N)],
        out_specs=_blk((RK, K), N // RK),
        out_shape=jax.ShapeDtypeStruct((B * N, K), _I),
    )(ca2, caT)
    idxf = idx.reshape(B * N * K, 1)

    hv = pl.pallas_call(
        _ctx_body,
        grid=(B, N // RC),
        in_specs=[_blk((RC, 3), N // RC), _perb3(3, M), _perb3(1, M),
                  _perb3(M, 1), _full((NE, 16)), _full((32, H)),
                  _full((1, H))],
        out_specs=_blk((RC, H), N // RC),
        out_shape=jax.ShapeDtypeStruct((B * N, H), _F),
    )(ca2, yT, ym3, ytc, elem_emb, ctx_W, ctx_b.reshape(1, H))

    E = pl.pallas_call(
        _edge_body,
        grid=(B, N // RE),
        in_specs=[_perb((N, 12)), _blk((RE, 12), N // RE),
                  _blk((RE * K, 1), N // RE), _full((321, H)), _full((1, H))],
        out_specs=_blk((RE * K, H), N // RE),
        out_shape=jax.ShapeDtypeStruct((B * N * K, H), _F),
    )(x12, x12, idxf, W_e, b_e.reshape(1, H))

    for l in range(NL):
        hv = pl.pallas_call(
            _layer_body,
            grid=(B, N // RL),
            in_specs=[_perb((N, H)), _blk((RL, H), N // RL),
                      _blk((RL * K, H), N // RL), _blk((RL * K, 1), N // RL),
                      _full((3 * H, H)), _full((1, H)), _full((H, H)),
                      _full((1, H)), _full((H, H)), _full((1, H)),
                      _full((1, H)), _full((1, H)), _full((H, 4 * H)),
                      _full((1, 4 * H)), _full((4 * H, H)), _full((1, H)),
                      _full((1, H)), _full((1, H)), _blk((RL, 1), N // RL)],
            out_specs=_blk((RL, H), N // RL),
            out_shape=jax.ShapeDtypeStruct((B * N, H), _F),
        )(hv, hv, E, idxf, L_W1[l], L_b1[l].reshape(1, H), L_W2[l],
          L_b2[l].reshape(1, H), L_W3[l], L_b3[l].reshape(1, H),
          L_n1g[l].reshape(1, H), L_n1b[l].reshape(1, H), L_Wi[l],
          L_bi[l].reshape(1, 4 * H), L_Wo[l], L_bo[l].reshape(1, H),
          L_n2g[l].reshape(1, H), L_n2b[l].reshape(1, H), maskc)

    out = pl.pallas_call(
        _final_body,
        grid=(B * N // RF, 1),
        in_specs=[_blk((RF, H), 1), _full((1, H)), _full((1, H)),
                  _full((H, CS)), _full((1, CS))],
        out_specs=_blk((RF, CS), 1),
        out_shape=jax.ShapeDtypeStruct((B * N, CS), _F),
    )(hv, fln_g.reshape(1, H), fln_b.reshape(1, H), Wd, bd.reshape(1, CS))
    return out.reshape(B, N, CS)


# speedup vs baseline: 7.0219x; 1.2079x over previous
import jax
import jax.numpy as jnp
from jax import lax
from jax.experimental import pallas as pl
from jax.experimental.pallas import tpu as pltpu

B = 2
N = 1024
M = 300
K = 48
C = 25
H = 128
CS = 384
NE = 64
NL = 3

RK = 256
RC = 256
RE = 64
RL = 64
RF = 256

_F = jnp.float32
_I = jnp.int32


def _centers(rows):
    i = lax.broadcasted_iota(_I, (rows, 16), 1).astype(_F)
    return 2.0 + i * (20.0 / 15.0)


def _rbf(d, rows):
    z = (d - _centers(rows)) / 1.25
    return jnp.exp(-(z * z))


def _knn_body(ca_ref, caT_ref, idx_ref):
    nb = pl.program_id(1)
    ca = ca_ref[...]
    x = ca[:, 0:1]
    y = ca[:, 1:2]
    z = ca[:, 2:3]
    caT = caT_ref[...][0]
    dx = x - caT[0:1, :]
    dy = y - caT[1:2, :]
    dz = z - caT[2:3, :]
    d = jnp.sqrt(dx * dx + dy * dy + dz * dz + 1e-6)
    rows = nb * RK + lax.broadcasted_iota(_I, (RK, N), 0)
    cols = lax.broadcasted_iota(_I, (RK, N), 1)
    d = jnp.where(rows == cols, d + 1e6, d)
    colsK = lax.broadcasted_iota(_I, (RK, K), 1)
    idxm = jnp.zeros((RK, K), _I)
    for k in range(K):
        m = jnp.min(d, axis=1, keepdims=True)
        am = jnp.min(jnp.where(d == m, cols, N), axis=1, keepdims=True)
        idxm = jnp.where(colsK == k, am, idxm)
        d = jnp.where(cols == am, 3e38, d)
    idx_ref[...] = idxm


def _ctx_body(ca_ref, yT_ref, ym_ref, ytc_ref, emb_ref, cw_ref, cb_ref, out_ref):
    ca = ca_ref[...]
    x = ca[:, 0:1]
    y = ca[:, 1:2]
    z = ca[:, 2:3]
    yT = yT_ref[...][0]
    dx = x - yT[0:1, :]
    dy = y - yT[1:2, :]
    dz = z - yT[2:3, :]
    d = jnp.sqrt(dx * dx + dy * dy + dz * dz + 1e-6)
    d = jnp.where(ym_ref[...][0] > 0, d, 1e6)
    colsM = lax.broadcasted_iota(_I, (RC, M), 1)
    ohy = (ytc_ref[...][0] == lax.broadcasted_iota(_I, (M, NE), 1)).astype(_F)
    emby = jnp.dot(ohy, emb_ref[...], preferred_element_type=_F)
    cw = cw_ref[...]
    cb = cb_ref[...]
    acc = jnp.zeros((RC, H), _F)
    for c in range(C):
        m = jnp.min(d, axis=1, keepdims=True)
        am = jnp.min(jnp.where(d == m, colsM, M), axis=1, keepdims=True)
        d = jnp.where(colsM == am, 3e6, d)
        r = _rbf(m, RC)
        oh = (colsM == am).astype(_F)
        e_c = jnp.dot(oh, emby, preferred_element_type=_F)
        pre = (jnp.dot(r, cw[0:16, :], preferred_element_type=_F)
               + jnp.dot(e_c, cw[16:32, :], preferred_element_type=_F) + cb)
        acc = acc + jax.nn.gelu(pre)
    out_ref[...] = acc / C


def _edge_body(x12f_ref, x12b_ref, idxf_ref, we_ref, be_ref, out_ref):
    nb = pl.program_id(1)
    EK = RE * K
    colsN = lax.broadcasted_iota(_I, (EK, N), 1)
    idxf = idxf_ref[...]
    oh = (idxf == colsN).astype(_F)
    xn = jnp.dot(oh, x12f_ref[...], preferred_element_type=_F)
    rep = ((lax.broadcasted_iota(_I, (EK, RE), 0) // K)
           == lax.broadcasted_iota(_I, (EK, RE), 1)).astype(_F)
    xs = jnp.dot(rep, x12b_ref[...], preferred_element_type=_F)
    d2 = None
    for c in range(3):
        r12 = lax.broadcasted_iota(_I, (12, 16), 0)
        p16 = lax.broadcasted_iota(_I, (12, 16), 1)
        sa = (r12 == 3 * (p16 // 4) + c).astype(_F)
        sb = (r12 == 3 * (p16 % 4) + c).astype(_F)
        t = (jnp.dot(xs, sa, preferred_element_type=_F)
             - jnp.dot(xn, sb, preferred_element_type=_F))
        t = t * t
        d2 = t if d2 is None else d2 + t
    d = jnp.sqrt(d2 + 1e-6)
    expand = (lax.broadcasted_iota(_I, (16, 256), 1) // 16
              == lax.broadcasted_iota(_I, (16, 256), 0)).astype(_F)
    d256 = jnp.dot(d, expand, preferred_element_type=_F)
    j256 = lax.broadcasted_iota(_I, (EK, 256), 1) % 16
    c256 = 2.0 + j256.astype(_F) * (20.0 / 15.0)
    z = (d256 - c256) * 0.8
    feats = jnp.exp(-(z * z))
    rowid = nb * RE + lax.broadcasted_iota(_I, (EK, 1), 0) // K
    off = jnp.clip(idxf - rowid, -32, 32) + 32
    ohoff = (off == lax.broadcasted_iota(_I, (EK, 65), 1)).astype(_F)
    we = we_ref[...]
    e = (jnp.dot(feats, we[0:256, :], preferred_element_type=_F)
         + jnp.dot(ohoff, we[256:321, :], preferred_element_type=_F)
         + be_ref[...])
    out_ref[...] = e


def _ln(h, g, b):
    mu = jnp.mean(h, axis=1, keepdims=True)
    v = jnp.mean((h - mu) ** 2, axis=1, keepdims=True)
    return (h - mu) / jnp.sqrt(v + 1e-5) * g + b


def _layer_body(hvf_ref, hvb_ref, eb_ref, idxf_ref, w1_ref, b1_ref, w2_ref,
                b2_ref, w3_ref, b3_ref, g1_ref, bb1_ref, wi_ref, bi_ref,
                wo_ref, bo_ref, g2_ref, bb2_ref, mk_ref, out_ref):
    EK = RL * K
    colsN = lax.broadcasted_iota(_I, (EK, N), 1)
    oh = (idxf_ref[...] == colsN).astype(jnp.bfloat16)
    hnb = jnp.dot(oh, hvf_ref[...].astype(jnp.bfloat16),
                  preferred_element_type=_F)
    rep = ((lax.broadcasted_iota(_I, (EK, RL), 0) // K)
           == lax.broadcasted_iota(_I, (EK, RL), 1)).astype(_F)
    hvb = hvb_ref[...]
    hvrep = jnp.dot(rep, hvb, preferred_element_type=_F)
    w1 = w1_ref[...]
    pre = (jnp.dot(hvrep, w1[0:H, :], preferred_element_type=_F)
           + jnp.dot(eb_ref[...], w1[H:2 * H, :], preferred_element_type=_F)
           + jnp.dot(hnb, w1[2 * H:3 * H, :], preferred_element_type=_F)
           + b1_ref[...])
    m = jax.nn.gelu(pre)
    m = jax.nn.gelu(jnp.dot(m, w2_ref[...], preferred_element_type=_F) + b2_ref[...])
    m = jnp.dot(m, w3_ref[...], preferred_element_type=_F) + b3_ref[...]
    summat = (lax.broadcasted_iota(_I, (RL, EK), 0)
              == lax.broadcasted_iota(_I, (RL, EK), 1) // K).astype(_F)
    msum = jnp.dot(summat, m, preferred_element_type=_F)
    h = hvb + msum / K
    hn = _ln(h, g1_ref[...], bb1_ref[...])
    ff = (jnp.dot(jax.nn.gelu(jnp.dot(hn, wi_ref[...], preferred_element_type=_F)
                              + bi_ref[...]),
                  wo_ref[...], preferred_element_type=_F) + bo_ref[...])
    h2 = _ln(hn + ff, g2_ref[...], bb2_ref[...])
    out_ref[...] = h2 * mk_ref[...]


def _final_body(hv_ref, g_ref, b_ref, wd_ref, bd_ref, out_ref):
    hn = _ln(hv_ref[...], g_ref[...], b_ref[...])
    out_ref[...] = jnp.dot(hn, wd_ref[...], preferred_element_type=_F) + bd_ref[...]


def _full(shape):
    return pl.BlockSpec(shape, lambda b, nb: (0, 0))


def _perb(shape):
    return pl.BlockSpec(shape, lambda b, nb: (b, 0))


def _perb3(d1, d2):
    return pl.BlockSpec((1, d1, d2), lambda b, nb: (b, 0, 0))


def _blk(shape, nblk):
    return pl.BlockSpec(shape, lambda b, nb: (b * nblk + nb, 0))


def kernel(X, Y, Y_m, mask, W_e, b_e, elem_emb, ctx_W, ctx_b, L_W1, L_b1,
           L_W2, L_b2, L_W3, L_b3, L_n1g, L_n1b, L_Wi, L_bi, L_Wo, L_bo,
           L_n2g, L_n2b, fln_g, fln_b, Wd, bd, Y_t, R_idx):
    Ca = X[:, :, 1, :]
    ca2 = Ca.reshape(B * N, 3)
    caT = jnp.swapaxes(Ca, 1, 2)
    x12 = X.reshape(B * N, 12)
    yT = jnp.swapaxes(Y, 1, 2)
    ym3 = Y_m.reshape(B, 1, M)
    ytc = Y_t.reshape(B, M, 1).astype(_I)
    maskc = mask.reshape(B * N, 1)

    idx = pl.pallas_call(
        _knn_body,
        grid=(B, N // RK),
        in_specs=[_blk((RK, 3), N // RK), _perb3(3, N)],
        out_specs=_blk((RK, K), N // RK),
        out_shape=jax.ShapeDtypeStruct((B * N, K), _I),
    )(ca2, caT)
    idxf = idx.reshape(B * N * K, 1)

    hv = pl.pallas_call(
        _ctx_body,
        grid=(B, N // RC),
        in_specs=[_blk((RC, 3), N // RC), _perb3(3, M), _perb3(1, M),
                  _perb3(M, 1), _full((NE, 16)), _full((32, H)),
                  _full((1, H))],
        out_specs=_blk((RC, H), N // RC),
        out_shape=jax.ShapeDtypeStruct((B * N, H), _F),
    )(ca2, yT, ym3, ytc, elem_emb, ctx_W, ctx_b.reshape(1, H))

    E = pl.pallas_call(
        _edge_body,
        grid=(B, N // RE),
        in_specs=[_perb((N, 12)), _blk((RE, 12), N // RE),
                  _blk((RE * K, 1), N // RE), _full((321, H)), _full((1, H))],
        out_specs=_blk((RE * K, H), N // RE),
        out_shape=jax.ShapeDtypeStruct((B * N * K, H), _F),
    )(x12, x12, idxf, W_e, b_e.reshape(1, H))

    for l in range(NL):
        hv = pl.pallas_call(
            _layer_body,
            grid=(B, N // RL),
            in_specs=[_perb((N, H)), _blk((RL, H), N // RL),
                      _blk((RL * K, H), N // RL), _blk((RL * K, 1), N // RL),
                      _full((3 * H, H)), _full((1, H)), _full((H, H)),
                      _full((1, H)), _full((H, H)), _full((1, H)),
                      _full((1, H)), _full((1, H)), _full((H, 4 * H)),
                      _full((1, 4 * H)), _full((4 * H, H)), _full((1, H)),
                      _full((1, H)), _full((1, H)), _blk((RL, 1), N // RL)],
            out_specs=_blk((RL, H), N // RL),
            out_shape=jax.ShapeDtypeStruct((B * N, H), _F),
        )(hv, hv, E, idxf, L_W1[l], L_b1[l].reshape(1, H), L_W2[l],
          L_b2[l].reshape(1, H), L_W3[l], L_b3[l].reshape(1, H),
          L_n1g[l].reshape(1, H), L_n1b[l].reshape(1, H), L_Wi[l],
          L_bi[l].reshape(1, 4 * H), L_Wo[l], L_bo[l].reshape(1, H),
          L_n2g[l].reshape(1, H), L_n2b[l].reshape(1, H), maskc)

    out = pl.pallas_call(
        _final_body,
        grid=(B * N // RF, 1),
        in_specs=[_blk((RF, H), 1), _full((1, H)), _full((1, H)),
                  _full((H, CS)), _full((1, CS))],
        out_specs=_blk((RF, CS), 1),
        out_shape=jax.ShapeDtypeStruct((B * N, CS), _F),
    )(hv, fln_g.reshape(1, H), fln_b.reshape(1, H), Wd, bd.reshape(1, CS))
    return out.reshape(B, N, CS)


# speedup vs baseline: 7.3395x; 1.0452x over previous
import jax
import jax.numpy as jnp
from jax import lax
from jax.experimental import pallas as pl
from jax.experimental.pallas import tpu as pltpu

B = 2
N = 1024
M = 300
K = 48
C = 25
H = 128
CS = 384
NE = 64
NL = 3

RK = 512
RC = 512
RE = 64
RL = 128
RF = 256

_F = jnp.float32
_I = jnp.int32


def _centers(rows):
    i = lax.broadcasted_iota(_I, (rows, 16), 1).astype(_F)
    return 2.0 + i * (20.0 / 15.0)


def _rbf(d, rows):
    z = (d - _centers(rows)) / 1.25
    return jnp.exp(-(z * z))


def _knn_body(ca_ref, caT_ref, idx_ref):
    nb = pl.program_id(1)
    ca = ca_ref[...]
    x = ca[:, 0:1]
    y = ca[:, 1:2]
    z = ca[:, 2:3]
    caT = caT_ref[...][0]
    dx = x - caT[0:1, :]
    dy = y - caT[1:2, :]
    dz = z - caT[2:3, :]
    d = jnp.sqrt(dx * dx + dy * dy + dz * dz + 1e-6)
    rows = nb * RK + lax.broadcasted_iota(_I, (RK, N), 0)
    cols = lax.broadcasted_iota(_I, (RK, N), 1)
    d = jnp.where(rows == cols, d + 1e6, d)
    colsK = lax.broadcasted_iota(_I, (RK, K), 1)
    idxm = jnp.zeros((RK, K), _I)
    for k in range(K):
        m = jnp.min(d, axis=1, keepdims=True)
        am = jnp.min(jnp.where(d == m, cols, N), axis=1, keepdims=True)
        idxm = jnp.where(colsK == k, am, idxm)
        d = jnp.where(cols == am, 3e38, d)
    idx_ref[...] = idxm


def _ctx_body(ca_ref, yT_ref, ym_ref, ytc_ref, emb_ref, cw_ref, cb_ref, out_ref):
    ca = ca_ref[...]
    x = ca[:, 0:1]
    y = ca[:, 1:2]
    z = ca[:, 2:3]
    yT = yT_ref[...][0]
    dx = x - yT[0:1, :]
    dy = y - yT[1:2, :]
    dz = z - yT[2:3, :]
    d = jnp.sqrt(dx * dx + dy * dy + dz * dz + 1e-6)
    d = jnp.where(ym_ref[...][0] > 0, d, 1e6)
    colsM = lax.broadcasted_iota(_I, (RC, M), 1)
    ohy = (ytc_ref[...][0] == lax.broadcasted_iota(_I, (M, NE), 1)).astype(_F)
    emby = jnp.dot(ohy, emb_ref[...], preferred_element_type=_F)
    cw = cw_ref[...]
    cb = cb_ref[...]
    acc = jnp.zeros((RC, H), _F)
    for c in range(C):
        m = jnp.min(d, axis=1, keepdims=True)
        am = jnp.min(jnp.where(d == m, colsM, M), axis=1, keepdims=True)
        d = jnp.where(colsM == am, 3e6, d)
        r = _rbf(m, RC)
        oh = (colsM == am).astype(_F)
        e_c = jnp.dot(oh, emby, preferred_element_type=_F)
        pre = (jnp.dot(r, cw[0:16, :], preferred_element_type=_F)
               + jnp.dot(e_c, cw[16:32, :], preferred_element_type=_F) + cb)
        acc = acc + jax.nn.gelu(pre)
    out_ref[...] = acc / C


def _edge_body(x12f_ref, x12b_ref, idxf_ref, we_ref, be_ref, out_ref):
    nb = pl.program_id(1)
    EK = RE * K
    colsN = lax.broadcasted_iota(_I, (EK, N), 1)
    idxf = idxf_ref[...]
    oh = (idxf == colsN).astype(_F)
    xn = jnp.dot(oh, x12f_ref[...], preferred_element_type=_F)
    rep = ((lax.broadcasted_iota(_I, (EK, RE), 0) // K)
           == lax.broadcasted_iota(_I, (EK, RE), 1)).astype(_F)
    xs = jnp.dot(rep, x12b_ref[...], preferred_element_type=_F)
    d2 = None
    for c in range(3):
        r12 = lax.broadcasted_iota(_I, (12, 16), 0)
        p16 = lax.broadcasted_iota(_I, (12, 16), 1)
        sa = (r12 == 3 * (p16 // 4) + c).astype(_F)
        sb = (r12 == 3 * (p16 % 4) + c).astype(_F)
        t = (jnp.dot(xs, sa, preferred_element_type=_F)
             - jnp.dot(xn, sb, preferred_element_type=_F))
        t = t * t
        d2 = t if d2 is None else d2 + t
    d = jnp.sqrt(d2 + 1e-6)
    expand = (lax.broadcasted_iota(_I, (16, 256), 1) // 16
              == lax.broadcasted_iota(_I, (16, 256), 0)).astype(_F)
    d256 = jnp.dot(d, expand, preferred_element_type=_F)
    j256 = lax.broadcasted_iota(_I, (EK, 256), 1) % 16
    c256 = 2.0 + j256.astype(_F) * (20.0 / 15.0)
    z = (d256 - c256) * 0.8
    feats = jnp.exp(-(z * z))
    rowid = nb * RE + lax.broadcasted_iota(_I, (EK, 1), 0) // K
    off = jnp.clip(idxf - rowid, -32, 32) + 32
    ohoff = (off == lax.broadcasted_iota(_I, (EK, 65), 1)).astype(_F)
    we = we_ref[...]
    e = (jnp.dot(feats, we[0:256, :], preferred_element_type=_F)
         + jnp.dot(ohoff, we[256:321, :], preferred_element_type=_F)
         + be_ref[...])
    out_ref[...] = e


def _ln(h, g, b):
    mu = jnp.mean(h, axis=1, keepdims=True)
    v = jnp.mean((h - mu) ** 2, axis=1, keepdims=True)
    return (h - mu) / jnp.sqrt(v + 1e-5) * g + b


def _layer_body(hvf_ref, hvb_ref, eb_ref, idxf_ref, w1_ref, b1_ref, w2_ref,
                b2_ref, w3_ref, b3_ref, g1_ref, bb1_ref, wi_ref, bi_ref,
                wo_ref, bo_ref, g2_ref, bb2_ref, mk_ref, out_ref):
    EK = RL * K
    colsN = lax.broadcasted_iota(_I, (EK, N), 1)
    oh = (idxf_ref[...] == colsN).astype(jnp.bfloat16)
    hnb = jnp.dot(oh, hvf_ref[...].astype(jnp.bfloat16),
                  preferred_element_type=_F)
    rep = ((lax.broadcasted_iota(_I, (EK, RL), 0) // K)
           == lax.broadcasted_iota(_I, (EK, RL), 1)).astype(_F)
    hvb = hvb_ref[...]
    hvrep = jnp.dot(rep, hvb, preferred_element_type=_F)
    w1 = w1_ref[...]
    pre = (jnp.dot(hvrep, w1[0:H, :], preferred_element_type=_F)
           + jnp.dot(eb_ref[...], w1[H:2 * H, :], preferred_element_type=_F)
           + jnp.dot(hnb, w1[2 * H:3 * H, :], preferred_element_type=_F)
           + b1_ref[...])
    m = jax.nn.gelu(pre)
    m = jax.nn.gelu(jnp.dot(m, w2_ref[...], preferred_element_type=_F) + b2_ref[...])
    m = jnp.dot(m, w3_ref[...], preferred_element_type=_F) + b3_ref[...]
    summat = (lax.broadcasted_iota(_I, (RL, EK), 0)
              == lax.broadcasted_iota(_I, (RL, EK), 1) // K).astype(_F)
    msum = jnp.dot(summat, m, preferred_element_type=_F)
    h = hvb + msum / K
    hn = _ln(h, g1_ref[...], bb1_ref[...])
    ff = (jnp.dot(jax.nn.gelu(jnp.dot(hn, wi_ref[...], preferred_element_type=_F)
                              + bi_ref[...]),
                  wo_ref[...], preferred_element_type=_F) + bo_ref[...])
    h2 = _ln(hn + ff, g2_ref[...], bb2_ref[...])
    out_ref[...] = h2 * mk_ref[...]


def _final_body(hv_ref, g_ref, b_ref, wd_ref, bd_ref, out_ref):
    hn = _ln(hv_ref[...], g_ref[...], b_ref[...])
    out_ref[...] = jnp.dot(hn, wd_ref[...], preferred_element_type=_F) + bd_ref[...]


def _full(shape):
    return pl.BlockSpec(shape, lambda b, nb: (0, 0))


def _perb(shape):
    return pl.BlockSpec(shape, lambda b, nb: (b, 0))


def _perb3(d1, d2):
    return pl.BlockSpec((1, d1, d2), lambda b, nb: (b, 0, 0))


def _blk(shape, nblk):
    return pl.BlockSpec(shape, lambda b, nb: (b * nblk + nb, 0))


def kernel(X, Y, Y_m, mask, W_e, b_e, elem_emb, ctx_W, ctx_b, L_W1, L_b1,
           L_W2, L_b2, L_W3, L_b3, L_n1g, L_n1b, L_Wi, L_bi, L_Wo, L_bo,
           L_n2g, L_n2b, fln_g, fln_b, Wd, bd, Y_t, R_idx):
    Ca = X[:, :, 1, :]
    ca2 = Ca.reshape(B * N, 3)
    caT = jnp.swapaxes(Ca, 1, 2)
    x12 = X.reshape(B * N, 12)
    yT = jnp.swapaxes(Y, 1, 2)
    ym3 = Y_m.reshape(B, 1, M)
    ytc = Y_t.reshape(B, M, 1).astype(_I)
    maskc = mask.reshape(B * N, 1)

    idx = pl.pallas_call(
        _knn_body,
        grid=(B, N // RK),
        in_specs=[_blk((RK, 3), N // RK), _perb3(3, N)],
        out_specs=_blk((RK, K), N // RK),
        out_shape=jax.ShapeDtypeStruct((B * N, K), _I),
    )(ca2, caT)
    idxf = idx.reshape(B * N * K, 1)

    hv = pl.pallas_call(
        _ctx_body,
        grid=(B, N // RC),
        in_specs=[_blk((RC, 3), N // RC), _perb3(3, M), _perb3(1, M),
                  _perb3(M, 1), _full((NE, 16)), _full((32, H)),
                  _full((1, H))],
        out_specs=_blk((RC, H), N // RC),
        out_shape=jax.ShapeDtypeStruct((B * N, H), _F),
    )(ca2, yT, ym3, ytc, elem_emb, ctx_W, ctx_b.reshape(1, H))

    E = pl.pallas_call(
        _edge_body,
        grid=(B, N // RE),
        in_specs=[_perb((N, 12)), _blk((RE, 12), N // RE),
                  _blk((RE * K, 1), N // RE), _full((321, H)), _full((1, H))],
        out_specs=_blk((RE * K, H), N // RE),
        out_shape=jax.ShapeDtypeStruct((B * N * K, H), _F),
    )(x12, x12, idxf, W_e, b_e.reshape(1, H))

    for l in range(NL):
        hv = pl.pallas_call(
            _layer_body,
            grid=(B, N // RL),
            in_specs=[_perb((N, H)), _blk((RL, H), N // RL),
                      _blk((RL * K, H), N // RL), _blk((RL * K, 1), N // RL),
                      _full((3 * H, H)), _full((1, H)), _full((H, H)),
                      _full((1, H)), _full((H, H)), _full((1, H)),
                      _full((1, H)), _full((1, H)), _full((H, 4 * H)),
                      _full((1, 4 * H)), _full((4 * H, H)), _full((1, H)),
                      _full((1, H)), _full((1, H)), _blk((RL, 1), N // RL)],
            out_specs=_blk((RL, H), N // RL),
            out_shape=jax.ShapeDtypeStruct((B * N, H), _F),
        )(hv, hv, E, idxf, L_W1[l], L_b1[l].reshape(1, H), L_W2[l],
          L_b2[l].reshape(1, H), L_W3[l], L_b3[l].reshape(1, H),
          L_n1g[l].reshape(1, H), L_n1b[l].reshape(1, H), L_Wi[l],
          L_bi[l].reshape(1, 4 * H), L_Wo[l], L_bo[l].reshape(1, H),
          L_n2g[l].reshape(1, H), L_n2b[l].reshape(1, H), maskc)

    out = pl.pallas_call(
        _final_body,
        grid=(B * N // RF, 1),
        in_specs=[_blk((RF, H), 1), _full((1, H)), _full((1, H)),
                  _full((H, CS)), _full((1, CS))],
        out_specs=_blk((RF, CS), 1),
        out_shape=jax.ShapeDtypeStruct((B * N, CS), _F),
    )(hv, fln_g.reshape(1, H), fln_b.reshape(1, H), Wd, bd.reshape(1, CS))
    return out.reshape(B, N, CS)


# speedup vs baseline: 7.6242x; 1.0388x over previous
import functools

import jax
import jax.numpy as jnp
from jax import lax
from jax.experimental import pallas as pl
from jax.experimental.pallas import tpu as pltpu
from jax.experimental.pallas import tpu_sc as plsc

B = 2
N = 1024
M = 300
K = 48
C = 25
H = 128
CS = 384
NE = 64
NL = 3

RK = 512
RC = 512
RE = 64
RL = 128
RF = 256

_F = jnp.float32
_I = jnp.int32


def _make_sc_gather(E, D):
    info = plsc.get_sparse_core_info()
    nc, ns = info.num_cores, info.num_subcores
    nw = nc * ns
    per_w = E // nw
    ch = 128
    n_ch = per_w // ch
    mesh = plsc.VectorSubcoreMesh(core_axis_name="c", subcore_axis_name="s")

    @functools.partial(
        pl.kernel, mesh=mesh,
        out_type=jax.ShapeDtypeStruct((E, D), _F),
        scratch_types=[
            pltpu.VMEM((ch,), _I),
            pltpu.VMEM((ch, D), _F),
            pltpu.SemaphoreType.DMA,
        ],
    )
    def gk(idx_hbm, table_hbm, out_hbm, idx_v, rows_v, sem):
        wid = lax.axis_index("s") * nc + lax.axis_index("c")
        base = wid * per_w
        for t in range(n_ch):
            off = base + t * ch
            pltpu.sync_copy(idx_hbm.at[pl.ds(off, ch)], idx_v)
            pltpu.async_copy(table_hbm.at[idx_v], rows_v, sem).wait()
            pltpu.sync_copy(rows_v, out_hbm.at[pl.ds(off, ch)])

    return gk


def _centers(rows):
    i = lax.broadcasted_iota(_I, (rows, 16), 1).astype(_F)
    return 2.0 + i * (20.0 / 15.0)


def _rbf(d, rows):
    z = (d - _centers(rows)) / 1.25
    return jnp.exp(-(z * z))


def _knn_body(ca_ref, caT_ref, idx_ref):
    nb = pl.program_id(1)
    ca = ca_ref[...]
    x = ca[:, 0:1]
    y = ca[:, 1:2]
    z = ca[:, 2:3]
    caT = caT_ref[...][0]
    dx = x - caT[0:1, :]
    dy = y - caT[1:2, :]
    dz = z - caT[2:3, :]
    d = jnp.sqrt(dx * dx + dy * dy + dz * dz + 1e-6)
    rows = nb * RK + lax.broadcasted_iota(_I, (RK, N), 0)
    cols = lax.broadcasted_iota(_I, (RK, N), 1)
    d = jnp.where(rows == cols, d + 1e6, d)
    colsK = lax.broadcasted_iota(_I, (RK, K), 1)
    idxm = jnp.zeros((RK, K), _I)
    for k in range(K):
        m = jnp.min(d, axis=1, keepdims=True)
        am = jnp.min(jnp.where(d == m, cols, N), axis=1, keepdims=True)
        idxm = jnp.where(colsK == k, am, idxm)
        d = jnp.where(cols == am, 3e38, d)
    idx_ref[...] = idxm


def _ctx_body(ca_ref, yT_ref, ym_ref, ytc_ref, emb_ref, cw_ref, cb_ref, out_ref):
    ca = ca_ref[...]
    x = ca[:, 0:1]
    y = ca[:, 1:2]
    z = ca[:, 2:3]
    yT = yT_ref[...][0]
    dx = x - yT[0:1, :]
    dy = y - yT[1:2, :]
    dz = z - yT[2:3, :]
    d = jnp.sqrt(dx * dx + dy * dy + dz * dz + 1e-6)
    d = jnp.where(ym_ref[...][0] > 0, d, 1e6)
    colsM = lax.broadcasted_iota(_I, (RC, M), 1)
    ohy = (ytc_ref[...][0] == lax.broadcasted_iota(_I, (M, NE), 1)).astype(_F)
    emby = jnp.dot(ohy, emb_ref[...], preferred_element_type=_F)
    cw = cw_ref[...]
    cb = cb_ref[...]
    acc = jnp.zeros((RC, H), _F)
    for c in range(C):
        m = jnp.min(d, axis=1, keepdims=True)
        am = jnp.min(jnp.where(d == m, colsM, M), axis=1, keepdims=True)
        d = jnp.where(colsM == am, 3e6, d)
        r = _rbf(m, RC)
        oh = (colsM == am).astype(_F)
        e_c = jnp.dot(oh, emby, preferred_element_type=_F)
        pre = (jnp.dot(r, cw[0:16, :], preferred_element_type=_F)
               + jnp.dot(e_c, cw[16:32, :], preferred_element_type=_F) + cb)
        acc = acc + jax.nn.gelu(pre)
    out_ref[...] = acc / C


def _edge_body(x12f_ref, x12b_ref, idxf_ref, we_ref, be_ref, out_ref):
    nb = pl.program_id(1)
    EK = RE * K
    idxf = idxf_ref[...]
    colsN = lax.broadcasted_iota(_I, (EK, N), 1)
    oh = (idxf == colsN).astype(_F)
    xn = jnp.dot(oh, x12f_ref[...], preferred_element_type=_F)
    rep = ((lax.broadcasted_iota(_I, (EK, RE), 0) // K)
           == lax.broadcasted_iota(_I, (EK, RE), 1)).astype(_F)
    xs = jnp.dot(rep, x12b_ref[...], preferred_element_type=_F)
    d2 = None
    for c in range(3):
        r12 = lax.broadcasted_iota(_I, (12, 16), 0)
        p16 = lax.broadcasted_iota(_I, (12, 16), 1)
        sa = (r12 == 3 * (p16 // 4) + c).astype(_F)
        sb = (r12 == 3 * (p16 % 4) + c).astype(_F)
        t = (jnp.dot(xs, sa, preferred_element_type=_F)
             - jnp.dot(xn, sb, preferred_element_type=_F))
        t = t * t
        d2 = t if d2 is None else d2 + t
    d = jnp.sqrt(d2 + 1e-6)
    expand = (lax.broadcasted_iota(_I, (16, 256), 1) // 16
              == lax.broadcasted_iota(_I, (16, 256), 0)).astype(_F)
    d256 = jnp.dot(d, expand, preferred_element_type=_F)
    j256 = lax.broadcasted_iota(_I, (EK, 256), 1) % 16
    c256 = 2.0 + j256.astype(_F) * (20.0 / 15.0)
    z = (d256 - c256) * 0.8
    feats = jnp.exp(-(z * z))
    rowid = nb * RE + lax.broadcasted_iota(_I, (EK, 1), 0) // K
    off = jnp.clip(idxf - rowid, -32, 32) + 32
    ohoff = (off == lax.broadcasted_iota(_I, (EK, 65), 1)).astype(_F)
    we = we_ref[...]
    e = (jnp.dot(feats, we[0:256, :], preferred_element_type=_F)
         + jnp.dot(ohoff, we[256:321, :], preferred_element_type=_F)
         + be_ref[...])
    out_ref[...] = e


def _ln(h, g, b):
    mu = jnp.mean(h, axis=1, keepdims=True)
    v = jnp.mean((h - mu) ** 2, axis=1, keepdims=True)
    return (h - mu) / jnp.sqrt(v + 1e-5) * g + b


def _layer_body(hnb_ref, hvb_ref, eb_ref, w1_ref, b1_ref, w2_ref,
                b2_ref, w3_ref, b3_ref, g1_ref, bb1_ref, wi_ref, bi_ref,
                wo_ref, bo_ref, g2_ref, bb2_ref, mk_ref, out_ref):
    EK = RL * K
    hnb = hnb_ref[...]
    rep = ((lax.broadcasted_iota(_I, (EK, RL), 0) // K)
           == lax.broadcasted_iota(_I, (EK, RL), 1)).astype(_F)
    hvb = hvb_ref[...]
    hvrep = jnp.dot(rep, hvb, preferred_element_type=_F)
    w1 = w1_ref[...]
    pre = (jnp.dot(hvrep, w1[0:H, :], preferred_element_type=_F)
           + jnp.dot(eb_ref[...], w1[H:2 * H, :], preferred_element_type=_F)
           + jnp.dot(hnb, w1[2 * H:3 * H, :], preferred_element_type=_F)
           + b1_ref[...])
    m = jax.nn.gelu(pre)
    m = jax.nn.gelu(jnp.dot(m, w2_ref[...], preferred_element_type=_F) + b2_ref[...])
    m = jnp.dot(m, w3_ref[...], preferred_element_type=_F) + b3_ref[...]
    summat = (lax.broadcasted_iota(_I, (RL, EK), 0)
              == lax.broadcasted_iota(_I, (RL, EK), 1) // K).astype(_F)
    msum = jnp.dot(summat, m, preferred_element_type=_F)
    h = hvb + msum / K
    hn = _ln(h, g1_ref[...], bb1_ref[...])
    ff = (jnp.dot(jax.nn.gelu(jnp.dot(hn, wi_ref[...], preferred_element_type=_F)
                              + bi_ref[...]),
                  wo_ref[...], preferred_element_type=_F) + bo_ref[...])
    h2 = _ln(hn + ff, g2_ref[...], bb2_ref[...])
    out_ref[...] = h2 * mk_ref[...]


def _final_body(hv_ref, g_ref, b_ref, wd_ref, bd_ref, out_ref):
    hn = _ln(hv_ref[...], g_ref[...], b_ref[...])
    out_ref[...] = jnp.dot(hn, wd_ref[...], preferred_element_type=_F) + bd_ref[...]


def _full(shape):
    return pl.BlockSpec(shape, lambda b, nb: (0, 0))


def _perb(shape):
    return pl.BlockSpec(shape, lambda b, nb: (b, 0))


def _perb3(d1, d2):
    return pl.BlockSpec((1, d1, d2), lambda b, nb: (b, 0, 0))


def _blk(shape, nblk):
    return pl.BlockSpec(shape, lambda b, nb: (b * nblk + nb, 0))


def kernel(X, Y, Y_m, mask, W_e, b_e, elem_emb, ctx_W, ctx_b, L_W1, L_b1,
           L_W2, L_b2, L_W3, L_b3, L_n1g, L_n1b, L_Wi, L_bi, L_Wo, L_bo,
           L_n2g, L_n2b, fln_g, fln_b, Wd, bd, Y_t, R_idx):
    Ca = X[:, :, 1, :]
    ca2 = Ca.reshape(B * N, 3)
    caT = jnp.swapaxes(Ca, 1, 2)
    x12 = X.reshape(B * N, 12)
    yT = jnp.swapaxes(Y, 1, 2)
    ym3 = Y_m.reshape(B, 1, M)
    ytc = Y_t.reshape(B, M, 1).astype(_I)
    maskc = mask.reshape(B * N, 1)

    idx = pl.pallas_call(
        _knn_body,
        grid=(B, N // RK),
        in_specs=[_blk((RK, 3), N // RK), _perb3(3, N)],
        out_specs=_blk((RK, K), N // RK),
        out_shape=jax.ShapeDtypeStruct((B * N, K), _I),
    )(ca2, caT)
    idxf = idx.reshape(B * N * K, 1)

    hv = pl.pallas_call(
        _ctx_body,
        grid=(B, N // RC),
        in_specs=[_blk((RC, 3), N // RC), _perb3(3, M), _perb3(1, M),
                  _perb3(M, 1), _full((NE, 16)), _full((32, H)),
                  _full((1, H))],
        out_specs=_blk((RC, H), N // RC),
        out_shape=jax.ShapeDtypeStruct((B * N, H), _F),
    )(ca2, yT, ym3, ytc, elem_emb, ctx_W, ctx_b.reshape(1, H))

    idxg = (idx.reshape(B, N * K)
            + (jnp.arange(B, dtype=_I) * N)[:, None]).reshape(B * N * K)

    E = pl.pallas_call(
        _edge_body,
        grid=(B, N // RE),
        in_specs=[_perb((N, 12)), _blk((RE, 12), N // RE),
                  _blk((RE * K, 1), N // RE), _full((321, H)), _full((1, H))],
        out_specs=_blk((RE * K, H), N // RE),
        out_shape=jax.ShapeDtypeStruct((B * N * K, H), _F),
    )(x12, x12, idxf, W_e, b_e.reshape(1, H))

    hv_gather = _make_sc_gather(B * N * K, H)
    for l in range(NL):
        hnb = hv_gather(idxg, hv)
        hv = pl.pallas_call(
            _layer_body,
            grid=(B, N // RL),
            in_specs=[_blk((RL * K, H), N // RL), _blk((RL, H), N // RL),
                      _blk((RL * K, H), N // RL),
                      _full((3 * H, H)), _full((1, H)), _full((H, H)),
                      _full((1, H)), _full((H, H)), _full((1, H)),
                      _full((1, H)), _full((1, H)), _full((H, 4 * H)),
                      _full((1, 4 * H)), _full((4 * H, H)), _full((1, H)),
                      _full((1, H)), _full((1, H)), _blk((RL, 1), N // RL)],
            out_specs=_blk((RL, H), N // RL),
            out_shape=jax.ShapeDtypeStruct((B * N, H), _F),
        )(hnb, hv, E, L_W1[l], L_b1[l].reshape(1, H), L_W2[l],
          L_b2[l].reshape(1, H), L_W3[l], L_b3[l].reshape(1, H),
          L_n1g[l].reshape(1, H), L_n1b[l].reshape(1, H), L_Wi[l],
          L_bi[l].reshape(1, 4 * H), L_Wo[l], L_bo[l].reshape(1, H),
          L_n2g[l].reshape(1, H), L_n2b[l].reshape(1, H), maskc)

    out = pl.pallas_call(
        _final_body,
        grid=(B * N // RF, 1),
        in_specs=[_blk((RF, H), 1), _full((1, H)), _full((1, H)),
                  _full((H, CS)), _full((1, CS))],
        out_specs=_blk((RF, CS), 1),
        out_shape=jax.ShapeDtypeStruct((B * N, CS), _F),
    )(hv, fln_g.reshape(1, H), fln_b.reshape(1, H), Wd, bd.reshape(1, CS))
    return out.reshape(B, N, CS)
